# Initial kernel scaffold; baseline (speedup 1.0000x reference)
import jax
import jax.numpy as jnp
from jax.experimental import pallas as pl


def kernel(pose, train_poses, W0, b0, W1, b1, W2, b2, W3, b3):
    raise NotImplementedError("write your pallas kernel here")


# speedup vs baseline: 1.2217x; 1.2217x over previous
import functools

import jax
import jax.numpy as jnp
import numpy as np
from jax.experimental import pallas as pl
from jax.experimental.pallas import tpu as pltpu

_B = 256
_K = 10000
_J = 21
_D = 4
_IN = _J * _D
_KB = 1024
_NB = 10
_KP = _KB * _NB
_NN = 5
_BIG = 1e30


def _acos(x):
    ax = jnp.abs(x)
    s = jnp.sqrt(1.0 - ax)
    p = jnp.float32(-0.0012624911)
    p = p * ax + jnp.float32(0.0066700901)
    p = p * ax - jnp.float32(0.0170881256)
    p = p * ax + jnp.float32(0.0308918810)
    p = p * ax - jnp.float32(0.0501743046)
    p = p * ax + jnp.float32(0.0889789874)
    p = p * ax - jnp.float32(0.2145988016)
    p = p * ax + jnp.float32(1.5707963050)
    r = s * p
    return jnp.where(x < 0, jnp.float32(np.pi) - r, r)


def _kern(posej_ref, poseflat_ref, trt_ref, w0_ref, b0_ref, w1_ref,
          b1_ref, w2_ref, b2_ref, w3_ref, b3_ref, mmt_ref, out_ref,
          pn_scr, top5_scr):
    p = posej_ref[...]
    ss = jnp.sum(p * p, axis=2, keepdims=True)
    pn_scr[...] = p * jax.lax.rsqrt(jnp.maximum(ss, 1e-24))

    top5_scr[...] = jnp.full((_B, 128), _BIG, jnp.float32)

    iota_kb = jax.lax.broadcasted_iota(jnp.int32, (_B, _KB), 1)
    iota_c = jax.lax.broadcasted_iota(jnp.int32, (_B, _KB + 128), 1)

    def kblock(kb, carry):
        def jbody(j, acc):
            t = trt_ref[kb, j]
            pj = pn_scr[j]
            d = (pj[:, 0:1] * t[0:1, :] + pj[:, 1:2] * t[1:2, :]
                 + pj[:, 2:3] * t[2:3, :] + pj[:, 3:4] * t[3:4, :])
            d = jnp.clip(d, -1.0 + 1e-6, 1.0 - 1e-6)
            return acc + _acos(d)

        dist = jax.lax.fori_loop(
            0, _J, jbody, jnp.zeros((_B, _KB), jnp.float32)) * 0.5
        valid = (kb * _KB + iota_kb) < _K
        dist = jnp.where(valid, dist, _BIG)

        cand = jnp.concatenate([top5_scr[...], dist], axis=1)
        for i in range(_NN):
            m = jnp.min(cand, axis=1, keepdims=True)
            idx = jnp.where(cand == m, iota_c, _KB + 128)
            first = jnp.min(idx, axis=1, keepdims=True)
            cand = jnp.where(iota_c == first, _BIG, cand)
            top5_scr[:, i:i + 1] = m
        return carry

    jax.lax.fori_loop(0, _NB, kblock, 0)

    x = poseflat_ref[...]
    ssf = jnp.dot(x * x, mmt_ref[...], preferred_element_type=jnp.float32)
    xn = x * jax.lax.rsqrt(jnp.maximum(ssf, 1e-24))
    h = jnp.dot(xn, w0_ref[...], preferred_element_type=jnp.float32)
    h = jnp.maximum(h + b0_ref[...], 0.0)
    h = jnp.dot(h, w1_ref[...], preferred_element_type=jnp.float32)
    h = jnp.maximum(h + b1_ref[...], 0.0)
    h = jnp.dot(h, w2_ref[...], preferred_element_type=jnp.float32)
    h = jnp.maximum(h + b2_ref[...], 0.0)
    pred = jnp.dot(h, w3_ref[...], preferred_element_type=jnp.float32)
    pred = pred + b3_ref[...]

    lane = jax.lax.broadcasted_iota(jnp.int32, (_B, 128), 1)
    t5 = top5_scr[...]
    dv = jnp.sum(jnp.where(lane < _NN, t5, 0.0), axis=1,
                 keepdims=True) * (1.0 / _NN)
    out_ref[...] = jnp.sum(jnp.abs(pred - dv), axis=0,
                           keepdims=True) * (1.0 / _B)


@functools.partial(jax.jit, static_argnums=())
def kernel(pose, train_poses, W0, b0, W1, b1, W2, b2, W3, b3):
    posej = jnp.transpose(pose, (1, 0, 2))
    poseflat = pose.reshape(_B, _IN)
    t = jnp.transpose(train_poses, (1, 2, 0))
    t = jnp.pad(t, ((0, 0), (0, 0), (0, _KP - _K)))
    trt = jnp.transpose(t.reshape(_J, _D, _NB, _KB), (2, 0, 1, 3))
    mmt = jnp.asarray(np.kron(np.eye(_J, dtype=np.float32),
                              np.ones((_D, _D), dtype=np.float32)))
    out = pl.pallas_call(
        _kern,
        out_shape=jax.ShapeDtypeStruct((1, 1), jnp.float32),
        scratch_shapes=[
            pltpu.VMEM((_J, _B, _D), jnp.float32),
            pltpu.VMEM((_B, 128), jnp.float32),
        ],
    )(posej, poseflat, trt, W0, b0.reshape(1, -1), W1, b1.reshape(1, -1),
      W2, b2.reshape(1, -1), W3, b3.reshape(1, 1), mmt)
    return out[0, 0]


# speedup vs baseline: 1.5025x; 1.2298x over previous
import functools

import jax
import jax.numpy as jnp
import numpy as np
from jax.experimental import pallas as pl
from jax.experimental.pallas import tpu as pltpu

_B = 256
_K = 10000
_J = 21
_D = 4
_IN = _J * _D
_KB = 1024
_NB = 10
_KP = _KB * _NB
_NN = 5
_BIG = 1e30


def _acos(x):
    ax = jnp.minimum(jnp.abs(x), 1.0 - 1e-6)
    s = jnp.sqrt(1.0 - ax)
    p = jnp.float32(-0.0187293)
    p = p * ax + jnp.float32(0.0742610)
    p = p * ax - jnp.float32(0.2121144)
    p = p * ax + jnp.float32(1.5707288)
    r = s * p
    return jnp.where(x < 0, jnp.float32(np.pi) - r, r)


def _kern(posej_ref, poseflat_ref, trt_ref, w0_ref, b0_ref, w1_ref,
          b1_ref, w2_ref, b2_ref, w3_ref, b3_ref, mmt_ref, out_ref,
          pn_scr, top5_scr):
    p = posej_ref[...]
    ss = jnp.sum(p * p, axis=2, keepdims=True)
    pn_scr[...] = p * jax.lax.rsqrt(jnp.maximum(ss, 1e-24))

    top5_scr[...] = jnp.full((_B, 128), _BIG, jnp.float32)

    iota_kb = jax.lax.broadcasted_iota(jnp.int32, (_B, _KB), 1)
    iota_c = jax.lax.broadcasted_iota(jnp.int32, (_B, _KB + 128), 1)

    def kblock(kb, carry):
        def jbody(j, acc):
            t = trt_ref[kb, j]
            pj = pn_scr[j]
            d = (pj[:, 0:1] * t[0:1, :] + pj[:, 1:2] * t[1:2, :]
                 + pj[:, 2:3] * t[2:3, :] + pj[:, 3:4] * t[3:4, :])
            return acc + _acos(d)

        dist = jax.lax.fori_loop(
            0, _J, jbody, jnp.zeros((_B, _KB), jnp.float32))
        valid = (kb * _KB + iota_kb) < _K
        dist = jnp.where(valid, dist, _BIG)

        cand = jnp.concatenate([top5_scr[...], dist], axis=1)
        for i in range(_NN):
            m = jnp.min(cand, axis=1, keepdims=True)
            idx = jnp.where(cand == m, iota_c, _KB + 128)
            first = jnp.min(idx, axis=1, keepdims=True)
            cand = jnp.where(iota_c == first, _BIG, cand)
            top5_scr[:, i:i + 1] = m
        return carry

    jax.lax.fori_loop(0, _NB, kblock, 0)

    x = poseflat_ref[...]
    ssf = jnp.dot(x * x, mmt_ref[...], preferred_element_type=jnp.float32)
    xn = x * jax.lax.rsqrt(jnp.maximum(ssf, 1e-24))
    h = jnp.dot(xn, w0_ref[...], preferred_element_type=jnp.float32)
    h = jnp.maximum(h + b0_ref[...], 0.0)
    h = jnp.dot(h, w1_ref[...], preferred_element_type=jnp.float32)
    h = jnp.maximum(h + b1_ref[...], 0.0)
    h = jnp.dot(h, w2_ref[...], preferred_element_type=jnp.float32)
    h = jnp.maximum(h + b2_ref[...], 0.0)
    pred = jnp.dot(h, w3_ref[...], preferred_element_type=jnp.float32)
    pred = pred + b3_ref[...]

    lane = jax.lax.broadcasted_iota(jnp.int32, (_B, 128), 1)
    t5 = top5_scr[...]
    dv = jnp.sum(jnp.where(lane < _NN, t5, 0.0), axis=1,
                 keepdims=True) * (0.5 / _NN)
    out_ref[...] = jnp.sum(jnp.abs(pred - dv), axis=0,
                           keepdims=True) * (1.0 / _B)


@functools.partial(jax.jit, static_argnums=())
def kernel(pose, train_poses, W0, b0, W1, b1, W2, b2, W3, b3):
    posej = jnp.transpose(pose, (1, 0, 2))
    poseflat = pose.reshape(_B, _IN)
    t = jnp.transpose(train_poses, (1, 2, 0))
    t = jnp.pad(t, ((0, 0), (0, 0), (0, _KP - _K)))
    trt = jnp.transpose(t.reshape(_J, _D, _NB, _KB), (2, 0, 1, 3))
    mmt = jnp.asarray(np.kron(np.eye(_J, dtype=np.float32),
                              np.ones((_D, _D), dtype=np.float32)))
    out = pl.pallas_call(
        _kern,
        out_shape=jax.ShapeDtypeStruct((1, 1), jnp.float32),
        scratch_shapes=[
            pltpu.VMEM((_J, _B, _D), jnp.float32),
            pltpu.VMEM((_B, 128), jnp.float32),
        ],
    )(posej, poseflat, trt, W0, b0.reshape(1, -1), W1, b1.reshape(1, -1),
      W2, b2.reshape(1, -1), W3, b3.reshape(1, 1), mmt)
    return out[0, 0]
